# Initial kernel scaffold; baseline (speedup 1.0000x reference)
import jax
import jax.numpy as jnp
from jax.experimental import pallas as pl


def kernel(emb, W1, b1, g1, be1, W2, b2, g2, be2, W3, b3, g3, be3, L1w, L1b, L2w, L2b, L3w, L3b, L4w, L4b, edge_index):
    raise NotImplementedError("write your pallas kernel here")


# speedup vs baseline: 29.0682x; 29.0682x over previous
import functools

import jax
import jax.numpy as jnp
from jax import lax
from jax.experimental import pallas as pl
from jax.experimental.pallas import tpu as pltpu
from jax.experimental.pallas import tpu_sc as plsc

N = 10000
E = 320000
F = 32
NC, NS = 2, 16
NW = NC * NS
CH = 80
EW = E // NW
NCHUNK = EW // CH
NP = N // NS
DW = 16

_mesh = plsc.VectorSubcoreMesh(core_axis_name="c", subcore_axis_name="s")


def _lrelu(x):
    return jnp.where(x >= 0, x, 0.1 * x)


@functools.partial(
    pl.kernel,
    out_type=jax.ShapeDtypeStruct((NC, N, DW), jnp.float32),
    mesh=_mesh,
    compiler_params=pltpu.CompilerParams(use_tc_tiling_on_sc=False),
    scratch_types=[
        pltpu.VMEM_SHARED((N, DW), jnp.float32),
        pltpu.VMEM((NCHUNK, CH), jnp.int32),
        pltpu.VMEM((CH, DW), jnp.float32),
        pltpu.VMEM((NP, DW), jnp.float32),
        pltpu.SemaphoreType.DMA((2,)),
    ],
)
def _deg_sc(eir, z16, out, acc, didx, valbuf, zbuf, sem):
    cid = lax.axis_index("c")
    sid = lax.axis_index("s")
    wid = sid * NC + cid

    pltpu.sync_copy(eir.at[1, pl.ds(wid * NCHUNK, NCHUNK)], didx)
    pltpu.sync_copy(z16, zbuf)
    pltpu.sync_copy(zbuf, acc.at[pl.ds(sid * NP, NP)])

    lanes = lax.broadcasted_iota(jnp.int32, (16,), 0)
    one0 = jnp.where(lanes == 0, 1.0, 0.0).astype(jnp.float32)

    @pl.loop(0, CH)
    def _fill(j):
        valbuf[j, :] = one0

    plsc.subcore_barrier()

    @pl.loop(0, NCHUNK)
    def _scat(c):
        b = lax.rem(c, 2)
        pltpu.async_copy(valbuf, acc.at[didx.at[c]], sem.at[b], add=True).wait()

    plsc.subcore_barrier()
    pltpu.sync_copy(acc.at[pl.ds(sid * NP, NP)], zbuf)
    pltpu.sync_copy(zbuf, out.at[cid, pl.ds(sid * NP, NP)])


@functools.partial(
    pl.kernel,
    out_type=jax.ShapeDtypeStruct((NC, N, F), jnp.float32),
    mesh=_mesh,
    compiler_params=pltpu.CompilerParams(use_tc_tiling_on_sc=False),
    scratch_types=[
        pltpu.VMEM_SHARED((N, F), jnp.float32),
        pltpu.VMEM((NCHUNK, CH), jnp.int32),
        pltpu.VMEM((NCHUNK, CH), jnp.int32),
        pltpu.VMEM((2, CH, F), jnp.float32),
        pltpu.VMEM((NP, F), jnp.float32),
        pltpu.SemaphoreType.DMA((2,)),
        pltpu.SemaphoreType.DMA((2,)),
    ],
)
def _mp_sc(y, eir, z32, out, acc, sidx, didx, rbuf, zbuf, gsem, ssem):
    cid = lax.axis_index("c")
    sid = lax.axis_index("s")
    wid = sid * NC + cid

    pltpu.sync_copy(eir.at[0, pl.ds(wid * NCHUNK, NCHUNK)], sidx)
    pltpu.sync_copy(eir.at[1, pl.ds(wid * NCHUNK, NCHUNK)], didx)
    pltpu.sync_copy(z32, zbuf)
    pltpu.sync_copy(zbuf, acc.at[pl.ds(sid * NP, NP)])
    plsc.subcore_barrier()

    pltpu.async_copy(y.at[sidx.at[0]], rbuf.at[0], gsem.at[0])
    pltpu.async_copy(y.at[sidx.at[1]], rbuf.at[1], gsem.at[1])

    @pl.loop(0, NCHUNK)
    def _pipe(c):
        b = lax.rem(c, 2)
        pltpu.make_async_copy(y.at[sidx.at[c]], rbuf.at[b], gsem.at[b]).wait()
        pltpu.async_copy(rbuf.at[b], acc.at[didx.at[c]], ssem.at[b], add=True).wait()

        @pl.when(c + 2 < NCHUNK)
        def _next():
            pltpu.async_copy(y.at[sidx.at[c + 2]], rbuf.at[b], gsem.at[b])

    plsc.subcore_barrier()
    pltpu.sync_copy(acc.at[pl.ds(sid * NP, NP)], zbuf)
    pltpu.sync_copy(zbuf, out.at[cid, pl.ds(sid * NP, NP)])


def _dinv_of(degp):
    deg = degp[0, :, 0:1] + degp[1, :, 0:1] + 2.0
    return lax.rsqrt(deg)


def _prep_tc(emb, W1, degp, y1, xw1):
    dinv = _dinv_of(degp[...])
    xw = jnp.dot(emb[...], W1[...], preferred_element_type=jnp.float32)
    xw1[...] = xw
    y1[...] = dinv * xw


def _mid_tc(p, xw, degp, bv, gv, bev, Wn, yn, xwn):
    dinv = _dinv_of(degp[...])
    h = dinv * (p[0] + p[1]) + 2.0 * dinv * dinv * xw[...] + bv[...]
    h = _lrelu(h)
    m = jnp.mean(h, axis=0, keepdims=True)
    v = jnp.mean((h - m) ** 2, axis=0, keepdims=True)
    h = (h - m) * lax.rsqrt(v + 1e-5) * gv[...] + bev[...]
    xwv = jnp.dot(h, Wn[...], preferred_element_type=jnp.float32)
    xwn[...] = xwv
    yn[...] = dinv * xwv


def _post_tc(p, xw, degp, bv, gv, bev, x3):
    dinv = _dinv_of(degp[...])
    h = dinv * (p[0] + p[1]) + 2.0 * dinv * dinv * xw[...] + bv[...]
    h = _lrelu(h)
    m = jnp.mean(h, axis=0, keepdims=True)
    v = jnp.mean((h - m) ** 2, axis=0, keepdims=True)
    x3[...] = (h - m) * lax.rsqrt(v + 1e-5) * gv[...] + bev[...]


KB = 12800
GK = (N * F) // KB


def _mlp_tc(x3f, L1w, L1b, L2w, L2b, L3w, L3b, L4w, L4b, out, accs):
    k = pl.program_id(0)

    @pl.when(k == 0)
    def _init():
        accs[...] = jnp.zeros((1, 128), jnp.float32)

    accs[...] += jnp.dot(x3f[...], L1w[...], preferred_element_type=jnp.float32)

    @pl.when(k == GK - 1)
    def _head():
        z = _lrelu(accs[...] + L1b[...])
        z = _lrelu(jnp.dot(z, L2w[...], preferred_element_type=jnp.float32) + L2b[...])
        z = _lrelu(jnp.dot(z, L3w[...], preferred_element_type=jnp.float32) + L3b[...])
        out[...] = jnp.dot(z, L4w[...], preferred_element_type=jnp.float32) + L4b[...]


def _full(shape):
    return pl.BlockSpec(shape, lambda k: (0,) * len(shape))


_mlp_call = pl.pallas_call(
    _mlp_tc,
    grid=(GK,),
    in_specs=[
        pl.BlockSpec((1, KB), lambda k: (0, k)),
        pl.BlockSpec((KB, 128), lambda k: (k, 0)),
        _full((1, 128)),
        _full((128, 64)),
        _full((1, 64)),
        _full((64, 32)),
        _full((1, 32)),
        _full((32, 1)),
        _full((1, 1)),
    ],
    out_specs=_full((1, 1)),
    out_shape=jax.ShapeDtypeStruct((1, 1), jnp.float32),
    scratch_shapes=[pltpu.VMEM((1, 128), jnp.float32)],
)


def kernel(emb, W1, b1, g1, be1, W2, b2, g2, be2, W3, b3, g3, be3,
           L1w, L1b, L2w, L2b, L3w, L3b, L4w, L4b, edge_index):
    f32 = jnp.float32
    eir = edge_index.reshape(2, NW * NCHUNK, CH)
    z16 = jnp.zeros((NP, DW), f32)
    z32 = jnp.zeros((NP, F), f32)

    degp = _deg_sc(eir, z16)

    prep = pl.pallas_call(
        _prep_tc,
        out_shape=(jax.ShapeDtypeStruct((N, F), f32),
                   jax.ShapeDtypeStruct((N, F), f32)),
    )
    y1, xw1 = prep(emb, W1, degp)

    mid = pl.pallas_call(
        _mid_tc,
        out_shape=(jax.ShapeDtypeStruct((N, F), f32),
                   jax.ShapeDtypeStruct((N, F), f32)),
    )
    post = pl.pallas_call(
        _post_tc,
        out_shape=jax.ShapeDtypeStruct((N, F), f32),
    )

    p1 = _mp_sc(y1, eir, z32)
    y2, xw2 = mid(p1, xw1, degp, b1.reshape(1, F), g1.reshape(1, F),
                  be1.reshape(1, F), W2)
    p2 = _mp_sc(y2, eir, z32)
    y3, xw3 = mid(p2, xw2, degp, b2.reshape(1, F), g2.reshape(1, F),
                  be2.reshape(1, F), W3)
    p3 = _mp_sc(y3, eir, z32)
    x3 = post(p3, xw3, degp, b3.reshape(1, F), g3.reshape(1, F),
              be3.reshape(1, F))

    x3f = x3.reshape(1, N * F)
    return _mlp_call(x3f, L1w, L1b.reshape(1, 128), L2w, L2b.reshape(1, 64),
                     L3w, L3b.reshape(1, 32), L4w, L4b.reshape(1, 1))


# speedup vs baseline: 30.0707x; 1.0345x over previous
import functools

import jax
import jax.numpy as jnp
from jax import lax
from jax.experimental import pallas as pl
from jax.experimental.pallas import tpu as pltpu
from jax.experimental.pallas import tpu_sc as plsc

N = 10000
E = 320000
F = 32
NC, NS = 2, 16
NW = NC * NS
CH = 128
NCHW = 79
EP = NW * NCHW * CH
NPAD = 10112
NP = NPAD // NS
DW = 16
NB = 8
LAG = 4

_mesh = plsc.VectorSubcoreMesh(core_axis_name="c", subcore_axis_name="s")
_sc_params = pltpu.CompilerParams(use_tc_tiling_on_sc=False)


def _lrelu(x):
    return jnp.where(x >= 0, x, 0.1 * x)


@functools.partial(
    pl.kernel,
    out_type=jax.ShapeDtypeStruct((NC, NPAD, DW), jnp.float32),
    mesh=_mesh,
    compiler_params=_sc_params,
    scratch_types=[
        pltpu.VMEM_SHARED((NPAD, DW), jnp.float32),
        pltpu.VMEM((NCHW, CH), jnp.int32),
        pltpu.VMEM((CH, DW), jnp.float32),
        pltpu.VMEM((NP, DW), jnp.float32),
        pltpu.SemaphoreType.DMA((NB,)),
    ],
)
def _deg_sc(eir, z16, out, acc, didx, valbuf, zbuf, sem):
    cid = lax.axis_index("c")
    sid = lax.axis_index("s")
    wid = sid * NC + cid

    pltpu.sync_copy(eir.at[1, pl.ds(wid * NCHW, NCHW)], didx)
    pltpu.sync_copy(z16, zbuf)
    pltpu.sync_copy(zbuf, acc.at[pl.ds(sid * NP, NP)])

    lanes = lax.broadcasted_iota(jnp.int32, (16,), 0)
    one0 = jnp.where(lanes == 0, 1.0, 0.0).astype(jnp.float32)

    @pl.loop(0, CH)
    def _fill(j):
        valbuf[j, :] = one0

    plsc.subcore_barrier()

    @pl.loop(0, NCHW)
    def _scat(c):
        b = lax.rem(c, NB)

        @pl.when(c >= NB)
        def _w():
            pltpu.make_async_copy(valbuf, acc.at[didx.at[c]], sem.at[b]).wait()

        pltpu.async_copy(valbuf, acc.at[didx.at[c]], sem.at[b], add=True)

    @pl.loop(NCHW - NB, NCHW)
    def _drain(c):
        b = lax.rem(c, NB)
        pltpu.make_async_copy(valbuf, acc.at[didx.at[c]], sem.at[b]).wait()

    plsc.subcore_barrier()
    pltpu.sync_copy(acc.at[pl.ds(sid * NP, NP)], zbuf)
    pltpu.sync_copy(zbuf, out.at[cid, pl.ds(sid * NP, NP)])


@functools.partial(
    pl.kernel,
    out_type=jax.ShapeDtypeStruct((NC, NPAD, F), jnp.float32),
    mesh=_mesh,
    compiler_params=_sc_params,
    scratch_types=[
        pltpu.VMEM_SHARED((NPAD, F), jnp.float32),
        pltpu.VMEM((NCHW, CH), jnp.int32),
        pltpu.VMEM((NCHW, CH), jnp.int32),
        pltpu.VMEM((NB, CH, F), jnp.float32),
        pltpu.VMEM((NP, F), jnp.float32),
        pltpu.SemaphoreType.DMA((NB,)),
        pltpu.SemaphoreType.DMA((NB,)),
    ],
)
def _mp_sc(y, eir, z32, out, acc, sidx, didx, rbuf, zbuf, gsem, ssem):
    cid = lax.axis_index("c")
    sid = lax.axis_index("s")
    wid = sid * NC + cid

    pltpu.sync_copy(eir.at[0, pl.ds(wid * NCHW, NCHW)], sidx)
    pltpu.sync_copy(eir.at[1, pl.ds(wid * NCHW, NCHW)], didx)
    pltpu.sync_copy(z32, zbuf)
    pltpu.sync_copy(zbuf, acc.at[pl.ds(sid * NP, NP)])
    plsc.subcore_barrier()

    @pl.loop(0, NCHW + LAG)
    def _pipe(i):
        @pl.when(i < NCHW)
        def _gather():
            b = lax.rem(i, NB)

            @pl.when(i >= NB)
            def _wait_prev_scatter():
                pltpu.make_async_copy(
                    rbuf.at[b], acc.at[didx.at[i - NB]], ssem.at[b]).wait()

            pltpu.async_copy(y.at[sidx.at[i]], rbuf.at[b], gsem.at[b])

        @pl.when(i >= LAG)
        def _scatter():
            c = i - LAG
            b = lax.rem(c, NB)
            pltpu.make_async_copy(y.at[sidx.at[c]], rbuf.at[b], gsem.at[b]).wait()
            pltpu.async_copy(rbuf.at[b], acc.at[didx.at[c]], ssem.at[b], add=True)

    @pl.loop(NCHW - NB, NCHW)
    def _drain(c):
        b = lax.rem(c, NB)
        pltpu.make_async_copy(rbuf.at[b], acc.at[didx.at[c]], ssem.at[b]).wait()

    plsc.subcore_barrier()
    pltpu.sync_copy(acc.at[pl.ds(sid * NP, NP)], zbuf)
    pltpu.sync_copy(zbuf, out.at[cid, pl.ds(sid * NP, NP)])


def _dinv_of(degp):
    deg = degp[0, 0:N, 0:1] + degp[1, 0:N, 0:1] + 2.0
    return lax.rsqrt(deg)


def _prep_tc(emb, W1, degp, y1, xw1):
    dinv = _dinv_of(degp)
    xw = jnp.dot(emb[...], W1[...], preferred_element_type=jnp.float32)
    xw1[...] = xw
    y1[0:N, :] = dinv * xw
    y1[N:NPAD, :] = jnp.zeros((NPAD - N, F), jnp.float32)


def _mid_tc(p, xw, degp, bv, gv, bev, Wn, yn, xwn):
    dinv = _dinv_of(degp)
    h = dinv * (p[0, 0:N, :] + p[1, 0:N, :]) + 2.0 * dinv * dinv * xw[...] + bv[...]
    h = _lrelu(h)
    m = jnp.mean(h, axis=0, keepdims=True)
    v = jnp.mean((h - m) ** 2, axis=0, keepdims=True)
    h = (h - m) * lax.rsqrt(v + 1e-5) * gv[...] + bev[...]
    xwv = jnp.dot(h, Wn[...], preferred_element_type=jnp.float32)
    xwn[...] = xwv
    yn[0:N, :] = dinv * xwv
    yn[N:NPAD, :] = jnp.zeros((NPAD - N, F), jnp.float32)


def _post_tc(p, xw, degp, bv, gv, bev, x3):
    dinv = _dinv_of(degp)
    h = dinv * (p[0, 0:N, :] + p[1, 0:N, :]) + 2.0 * dinv * dinv * xw[...] + bv[...]
    h = _lrelu(h)
    m = jnp.mean(h, axis=0, keepdims=True)
    v = jnp.mean((h - m) ** 2, axis=0, keepdims=True)
    x3[...] = (h - m) * lax.rsqrt(v + 1e-5) * gv[...] + bev[...]


KB = 12800
GK = (N * F) // KB


def _mlp_tc(x3f, L1w, L1b, L2w, L2b, L3w, L3b, L4w, L4b, out, accs):
    k = pl.program_id(0)

    @pl.when(k == 0)
    def _init():
        accs[...] = jnp.zeros((1, 128), jnp.float32)

    accs[...] += jnp.dot(x3f[...], L1w[...], preferred_element_type=jnp.float32)

    @pl.when(k == GK - 1)
    def _head():
        z = _lrelu(accs[...] + L1b[...])
        z = _lrelu(jnp.dot(z, L2w[...], preferred_element_type=jnp.float32) + L2b[...])
        z = _lrelu(jnp.dot(z, L3w[...], preferred_element_type=jnp.float32) + L3b[...])
        out[...] = jnp.dot(z, L4w[...], preferred_element_type=jnp.float32) + L4b[...]


def _full(shape):
    return pl.BlockSpec(shape, lambda k: (0,) * len(shape))


_mlp_call = pl.pallas_call(
    _mlp_tc,
    grid=(GK,),
    in_specs=[
        pl.BlockSpec((1, KB), lambda k: (0, k)),
        pl.BlockSpec((KB, 128), lambda k: (k, 0)),
        _full((1, 128)),
        _full((128, 64)),
        _full((1, 64)),
        _full((64, 32)),
        _full((1, 32)),
        _full((32, 1)),
        _full((1, 1)),
    ],
    out_specs=_full((1, 1)),
    out_shape=jax.ShapeDtypeStruct((1, 1), jnp.float32),
    scratch_shapes=[pltpu.VMEM((1, 128), jnp.float32)],
)


def kernel(emb, W1, b1, g1, be1, W2, b2, g2, be2, W3, b3, g3, be3,
           L1w, L1b, L2w, L2b, L3w, L3b, L4w, L4b, edge_index):
    f32 = jnp.float32
    pad = jnp.full((2, EP - E), N, jnp.int32)
    eir = jnp.concatenate([edge_index, pad], axis=1).reshape(2, NW * NCHW, CH)
    z16 = jnp.zeros((NP, DW), f32)
    z32 = jnp.zeros((NP, F), f32)

    degp = _deg_sc(eir, z16)

    prep = pl.pallas_call(
        _prep_tc,
        out_shape=(jax.ShapeDtypeStruct((NPAD, F), f32),
                   jax.ShapeDtypeStruct((N, F), f32)),
    )
    y1, xw1 = prep(emb, W1, degp)

    mid = pl.pallas_call(
        _mid_tc,
        out_shape=(jax.ShapeDtypeStruct((NPAD, F), f32),
                   jax.ShapeDtypeStruct((N, F), f32)),
    )
    post = pl.pallas_call(
        _post_tc,
        out_shape=jax.ShapeDtypeStruct((N, F), f32),
    )

    p1 = _mp_sc(y1, eir, z32)
    y2, xw2 = mid(p1, xw1, degp, b1.reshape(1, F), g1.reshape(1, F),
                  be1.reshape(1, F), W2)
    p2 = _mp_sc(y2, eir, z32)
    y3, xw3 = mid(p2, xw2, degp, b2.reshape(1, F), g2.reshape(1, F),
                  be2.reshape(1, F), W3)
    p3 = _mp_sc(y3, eir, z32)
    x3 = post(p3, xw3, degp, b3.reshape(1, F), g3.reshape(1, F),
              be3.reshape(1, F))

    x3f = x3.reshape(1, N * F)
    return _mlp_call(x3f, L1w, L1b.reshape(1, 128), L2w, L2b.reshape(1, 64),
                     L3w, L3b.reshape(1, 32), L4w, L4b.reshape(1, 1))


# speedup vs baseline: 52.3842x; 1.7420x over previous
import functools

import jax
import jax.numpy as jnp
from jax import lax
from jax.experimental import pallas as pl
from jax.experimental.pallas import tpu as pltpu
from jax.experimental.pallas import tpu_sc as plsc

N = 10000
E = 320000
F = 32
NPK = N // 4
NC, NS = 2, 16
NW = NC * NS
CH = 128
TCH = E // CH
CW0 = TCH // NW
NXW = TCH - NW * CW0
CWMAX = CW0 + 1
NPAD = 10112
NP = NPAD // NS
NB = 8
LAG = 4

_mesh = plsc.VectorSubcoreMesh(core_axis_name="c", subcore_axis_name="s")
_sc_params = pltpu.CompilerParams(use_tc_tiling_on_sc=False)


def _lrelu(x):
    return jnp.where(x >= 0, x, 0.1 * x)


def _chunks_of(wid):
    nch = jnp.where(wid < NW - NXW, CW0, CWMAX)
    base = jnp.where(wid < NW - NXW, CW0 * wid,
                     CW0 * (NW - NXW) + CWMAX * (wid - (NW - NXW)))
    return nch, base


@functools.partial(
    pl.kernel,
    out_type=jax.ShapeDtypeStruct((NC, NPAD, F), jnp.float32),
    mesh=_mesh,
    compiler_params=_sc_params,
    scratch_types=[
        pltpu.VMEM_SHARED((NPAD, F), jnp.float32),
        pltpu.VMEM((CWMAX, CH), jnp.int32),
        pltpu.VMEM((CH, F), jnp.float32),
        pltpu.VMEM((NP, F), jnp.float32),
        pltpu.SemaphoreType.DMA((NB,)),
    ],
)
def _deg_sc(eir, z32, out, acc, didx, valbuf, zbuf, sem):
    cid = lax.axis_index("c")
    sid = lax.axis_index("s")
    wid = sid * NC + cid
    nch, base = _chunks_of(wid)

    pltpu.sync_copy(eir.at[1, pl.ds(base, CWMAX)], didx)
    pltpu.sync_copy(z32, zbuf)
    pltpu.sync_copy(zbuf, acc.at[pl.ds(sid * NP, NP)])

    lanes = lax.broadcasted_iota(jnp.int32, (16,), 0)
    one0 = jnp.where(lanes == 0, 1.0, 0.0).astype(jnp.float32)
    zero16 = jnp.zeros((16,), jnp.float32)

    @pl.loop(0, CH)
    def _fill(j):
        valbuf[j, 0:16] = one0
        valbuf[j, 16:32] = zero16

    plsc.subcore_barrier()

    @pl.loop(0, nch)
    def _scat(c):
        b = lax.rem(c, NB)

        @pl.when(c >= NB)
        def _w():
            pltpu.make_async_copy(valbuf, acc.at[didx.at[c]], sem.at[b]).wait()

        pltpu.async_copy(valbuf, acc.at[didx.at[c]], sem.at[b], add=True)

    @pl.loop(nch - NB, nch)
    def _drain(c):
        b = lax.rem(c, NB)
        pltpu.make_async_copy(valbuf, acc.at[didx.at[c]], sem.at[b]).wait()

    plsc.subcore_barrier()
    pltpu.sync_copy(acc.at[pl.ds(sid * NP, NP)], zbuf)
    pltpu.sync_copy(zbuf, out.at[cid, pl.ds(sid * NP, NP)])


@functools.partial(
    pl.kernel,
    out_type=jax.ShapeDtypeStruct((NC, NPAD, F), jnp.float32),
    mesh=_mesh,
    compiler_params=_sc_params,
    scratch_types=[
        pltpu.VMEM_SHARED((NPAD, F), jnp.float32),
        pltpu.VMEM((CWMAX, CH), jnp.int32),
        pltpu.VMEM((CWMAX, CH), jnp.int32),
        pltpu.VMEM((NB, CH, F), jnp.float32),
        pltpu.VMEM((NP, F), jnp.float32),
        pltpu.SemaphoreType.DMA((NB,)),
        pltpu.SemaphoreType.DMA((NB,)),
    ],
)
def _mp_sc(y, eir, z32, out, acc, sidx, didx, rbuf, zbuf, gsem, ssem):
    cid = lax.axis_index("c")
    sid = lax.axis_index("s")
    wid = sid * NC + cid
    nch, base = _chunks_of(wid)

    pltpu.sync_copy(eir.at[0, pl.ds(base, CWMAX)], sidx)
    pltpu.sync_copy(eir.at[1, pl.ds(base, CWMAX)], didx)
    pltpu.sync_copy(z32, zbuf)
    pltpu.sync_copy(zbuf, acc.at[pl.ds(sid * NP, NP)])
    plsc.subcore_barrier()

    @pl.loop(0, nch + LAG)
    def _pipe(i):
        @pl.when(i < nch)
        def _gather():
            b = lax.rem(i, NB)

            @pl.when(i >= NB)
            def _wait_prev_scatter():
                pltpu.make_async_copy(
                    rbuf.at[b], acc.at[didx.at[i - NB]], ssem.at[b]).wait()

            pltpu.async_copy(y.at[sidx.at[i]], rbuf.at[b], gsem.at[b])

        @pl.when(i >= LAG)
        def _scatter():
            c = i - LAG
            b = lax.rem(c, NB)
            pltpu.make_async_copy(y.at[sidx.at[c]], rbuf.at[b], gsem.at[b]).wait()
            pltpu.async_copy(rbuf.at[b], acc.at[didx.at[c]], ssem.at[b], add=True)

    @pl.loop(nch - NB, nch)
    def _drain(c):
        b = lax.rem(c, NB)
        pltpu.make_async_copy(rbuf.at[b], acc.at[didx.at[c]], ssem.at[b]).wait()

    plsc.subcore_barrier()
    pltpu.sync_copy(acc.at[pl.ds(sid * NP, NP)], zbuf)
    pltpu.sync_copy(zbuf, out.at[cid, pl.ds(sid * NP, NP)])


def _blockdiag4(W, rows, cols):
    z = jnp.zeros((rows, cols), jnp.float32)
    out_rows = []
    for i in range(4):
        parts = [W if j == i else z for j in range(4)]
        out_rows.append(jnp.concatenate(parts, axis=1))
    return jnp.concatenate(out_rows, axis=0)


def _bcast_mat():
    i0 = lax.broadcasted_iota(jnp.int32, (F, F), 0)
    e = jnp.where(i0 == 0, 1.0, 0.0).astype(jnp.float32)
    return _blockdiag4(e, F, F)


def _fold4(v):
    s = v[:, 0:32] + v[:, 32:64] + v[:, 64:96] + v[:, 96:128]
    s = 0.25 * s
    return jnp.concatenate([s, s, s, s], axis=1)


def _prep_tc(emb4, W1, degp, dinv, y1, xw1):
    d = degp[0, 0:NPK, :] + degp[1, 0:NPK, :]
    dv = lax.rsqrt(jnp.dot(d, _bcast_mat(), preferred_element_type=jnp.float32)
                   + 2.0)
    W1s = _blockdiag4(W1[...], 128, F)
    xw = jnp.dot(emb4[...], W1s, preferred_element_type=jnp.float32)
    dinv[...] = dv
    xw1[...] = xw
    y1[...] = dv * xw


def _mid_tc(p, xw, dinv, bv, gv, bev, Wn, yn, xwn):
    dv = dinv[...]
    h = dv * (p[0, 0:NPK, :] + p[1, 0:NPK, :]) + 2.0 * dv * dv * xw[...] + bv[...]
    h = _lrelu(h)
    m = _fold4(jnp.mean(h, axis=0, keepdims=True))
    v = _fold4(jnp.mean((h - m) ** 2, axis=0, keepdims=True))
    h = (h - m) * lax.rsqrt(v + 1e-5) * gv[...] + bev[...]
    xwv = jnp.dot(h, _blockdiag4(Wn[...], F, F), preferred_element_type=jnp.float32)
    xwn[...] = xwv
    yn[...] = dv * xwv


def _post_tc(p, xw, dinv, bv, gv, bev, x3):
    dv = dinv[...]
    h = dv * (p[0, 0:NPK, :] + p[1, 0:NPK, :]) + 2.0 * dv * dv * xw[...] + bv[...]
    h = _lrelu(h)
    m = _fold4(jnp.mean(h, axis=0, keepdims=True))
    v = _fold4(jnp.mean((h - m) ** 2, axis=0, keepdims=True))
    x3[...] = (h - m) * lax.rsqrt(v + 1e-5) * gv[...] + bev[...]


KB = 12800
GK = (N * F) // KB


def _mlp_tc(x3f, L1w, L1b, L2w, L2b, L3w, L3b, L4w, L4b, out, accs):
    k = pl.program_id(0)

    @pl.when(k == 0)
    def _init():
        accs[...] = jnp.zeros((1, 128), jnp.float32)

    accs[...] += jnp.dot(x3f[...], L1w[...], preferred_element_type=jnp.float32)

    @pl.when(k == GK - 1)
    def _head():
        z = _lrelu(accs[...] + L1b[...])
        z = _lrelu(jnp.dot(z, L2w[...], preferred_element_type=jnp.float32) + L2b[...])
        z = _lrelu(jnp.dot(z, L3w[...], preferred_element_type=jnp.float32) + L3b[...])
        out[...] = jnp.dot(z, L4w[...], preferred_element_type=jnp.float32) + L4b[...]


def _full(shape):
    return pl.BlockSpec(shape, lambda k: (0,) * len(shape))


_mlp_call = pl.pallas_call(
    _mlp_tc,
    grid=(GK,),
    in_specs=[
        pl.BlockSpec((1, KB), lambda k: (0, k)),
        pl.BlockSpec((KB, 128), lambda k: (k, 0)),
        _full((1, 128)),
        _full((128, 64)),
        _full((1, 64)),
        _full((64, 32)),
        _full((1, 32)),
        _full((32, 1)),
        _full((1, 1)),
    ],
    out_specs=_full((1, 1)),
    out_shape=jax.ShapeDtypeStruct((1, 1), jnp.float32),
    scratch_shapes=[pltpu.VMEM((1, 128), jnp.float32)],
)


def _pk(v):
    return jnp.tile(v, 4).reshape(1, 128)


def kernel(emb, W1, b1, g1, be1, W2, b2, g2, be2, W3, b3, g3, be3,
           L1w, L1b, L2w, L2b, L3w, L3b, L4w, L4b, edge_index):
    f32 = jnp.float32
    eir = edge_index.reshape(2, TCH, CH)
    z32 = jnp.zeros((NP, F), f32)
    emb4 = emb.reshape(NPK, 512)

    degp = _deg_sc(eir, z32).reshape(NC, NPAD // 4, 128)

    prep = pl.pallas_call(
        _prep_tc,
        out_shape=(jax.ShapeDtypeStruct((NPK, 128), f32),
                   jax.ShapeDtypeStruct((NPK, 128), f32),
                   jax.ShapeDtypeStruct((NPK, 128), f32)),
    )
    dinv, y1, xw1 = prep(emb4, W1, degp)

    mid = pl.pallas_call(
        _mid_tc,
        out_shape=(jax.ShapeDtypeStruct((NPK, 128), f32),
                   jax.ShapeDtypeStruct((NPK, 128), f32)),
    )
    post = pl.pallas_call(
        _post_tc,
        out_shape=jax.ShapeDtypeStruct((NPK, 128), f32),
    )

    p1 = _mp_sc(y1.reshape(N, F), eir, z32).reshape(NC, NPAD // 4, 128)
    y2, xw2 = mid(p1, xw1, dinv, _pk(b1), _pk(g1), _pk(be1), W2)
    p2 = _mp_sc(y2.reshape(N, F), eir, z32).reshape(NC, NPAD // 4, 128)
    y3, xw3 = mid(p2, xw2, dinv, _pk(b2), _pk(g2), _pk(be2), W3)
    p3 = _mp_sc(y3.reshape(N, F), eir, z32).reshape(NC, NPAD // 4, 128)
    x3 = post(p3, xw3, dinv, _pk(b3), _pk(g3), _pk(be3))

    x3f = x3.reshape(1, N * F)
    return _mlp_call(x3f, L1w, L1b.reshape(1, 128), L2w, L2b.reshape(1, 64),
                     L3w, L3b.reshape(1, 32), L4w, L4b.reshape(1, 1))
